# baseline (speedup 1.0000x reference)
import functools

import jax
import jax.numpy as jnp
from jax import lax
from jax.experimental import pallas as pl
from jax.experimental.pallas import tpu as pltpu
from jax.experimental.pallas import tpu_sc as plsc

NC, NS, L = 2, 16, 16
NW = NC * NS
B = 16384
D = 64
RPW = B // NW
CH = 128
NCHUNK = RPW // CH
CPAD = 1024

_EPS = 1e-12


def _rsqrt_vec(x):
    i = plsc.bitcast(x, jnp.int32)
    y = plsc.bitcast(jnp.int32(0x5F3759DF) - (i >> 1), jnp.float32)
    hx = x * jnp.float32(0.5)
    for _ in range(3):
        y = y * (jnp.float32(1.5) - (hx * y) * y)
    return y


def _prep_body(d_ref, r_ref, e_ref, c_ref):
    d = d_ref[...]
    r = r_ref[...]
    dn = jnp.maximum(jnp.sqrt(jnp.sum(d * d, axis=1, keepdims=True)), _EPS)
    rn = jnp.maximum(jnp.sqrt(jnp.sum(r * r, axis=1, keepdims=True)), _EPS)
    e = r / rn - d / dn
    e_ref[...] = e
    c_ref[...] = jnp.sum(e * e, axis=1)


@functools.partial(
    pl.kernel,
    out_type=jax.ShapeDtypeStruct((B,), jnp.float32),
    mesh=plsc.VectorSubcoreMesh(core_axis_name="c", subcore_axis_name="s"),
    scratch_types=[
        pltpu.VMEM((RPW,), jnp.int32),
        pltpu.VMEM((RPW,), jnp.int32),
        pltpu.VMEM((RPW, D), jnp.float32),
        pltpu.VMEM((RPW, D), jnp.float32),
        pltpu.VMEM((CPAD,), jnp.float32),
        pltpu.VMEM((RPW,), jnp.float32),
        pltpu.SemaphoreType.DMA,
    ],
    compiler_params=pltpu.CompilerParams(needs_layout_passes=False,
                                         use_tc_tiling_on_sc=False),
)
def _sc_main(gene_idx, drug_idx, gene_tab, e_tab, c_tab, out_hbm,
             gidx_v, didx_v, g_v, e_v, c_v, out_v, sem):
    wid = lax.axis_index("s") * NC + lax.axis_index("c")
    base = wid * RPW
    with jax.named_scope("idx_copy"):
        pltpu.sync_copy(gene_idx.at[pl.ds(base, RPW)], gidx_v)
        pltpu.sync_copy(drug_idx.at[pl.ds(base, RPW)], didx_v)
        pltpu.sync_copy(c_tab, c_v)
    with jax.named_scope("gather_fire"):
        copies = []
        for k in range(NCHUNK):
            copies.append(pltpu.async_copy(
                gene_tab.at[gidx_v.at[pl.ds(k * CH, CH)]],
                g_v.at[pl.ds(k * CH, CH)], sem))
            copies.append(pltpu.async_copy(
                e_tab.at[didx_v.at[pl.ds(k * CH, CH)]],
                e_v.at[pl.ds(k * CH, CH)], sem))
    with jax.named_scope("gather_wait"):
        for cp in copies:
            cp.wait()

    iota = lax.iota(jnp.int32, L)

    def grp(t, carry):
        rows = t * L + iota
        gg = jnp.zeros((L,), jnp.float32)
        ge = jnp.zeros((L,), jnp.float32)
        for j in range(D):
            cols = (iota + j) & (D - 1)
            g = plsc.load_gather(g_v, [rows, cols])
            e = plsc.load_gather(e_v, [rows, cols])
            gg = gg + g * g
            ge = ge + g * e
        di = didx_v[pl.ds(t * L, L)]
        cc = plsc.load_gather(c_v, [di])
        s = jnp.maximum(gg * _rsqrt_vec(gg), jnp.float32(_EPS))
        inv = jnp.float32(1.0) / s
        tot = jnp.maximum(gg * inv * inv + (jnp.float32(2.0) * inv) * ge + cc,
                          jnp.float32(0.0))
        out_v[pl.ds(t * L, L)] = jnp.float32(0.0) - tot * _rsqrt_vec(tot)
        return carry

    with jax.named_scope("compute"):
        lax.fori_loop(0, RPW // L, grp, 0)
    with jax.named_scope("writeback"):
        pltpu.sync_copy(out_v, out_hbm.at[pl.ds(base, RPW)])


def kernel(gene_idx, drug_idx, gene_table, drug_table, relation):
    gene_idx = gene_idx.astype(jnp.int32)
    drug_idx = drug_idx.astype(jnp.int32)
    nd = drug_table.shape[0]
    d_pad = jnp.pad(drug_table, ((0, CPAD - nd), (0, 0)), constant_values=1.0)
    e_tab, c_tab = pl.pallas_call(
        _prep_body,
        out_shape=[
            jax.ShapeDtypeStruct((CPAD, D), jnp.float32),
            jax.ShapeDtypeStruct((CPAD,), jnp.float32),
        ],
    )(d_pad, relation.reshape(1, D))
    return _sc_main(gene_idx, drug_idx, gene_table, e_tab, c_tab)


# speedup vs baseline: 2.2306x; 2.2306x over previous
import functools

import jax
import jax.numpy as jnp
from jax import lax
from jax.experimental import pallas as pl
from jax.experimental.pallas import tpu as pltpu
from jax.experimental.pallas import tpu_sc as plsc

NC, NS, L = 2, 16, 16
NW = NC * NS
B = 16384
D = 64
RPW = B // NW
CH = 64
NP = RPW // CH
CPAD = 1024
EW = 128

_EPS = 1e-12


def _rsqrt_vec(x):
    i = plsc.bitcast(x, jnp.int32)
    y = plsc.bitcast(jnp.int32(0x5F3759DF) - (i >> 1), jnp.float32)
    hx = x * jnp.float32(0.5)
    for _ in range(3):
        y = y * (jnp.float32(1.5) - (hx * y) * y)
    return y


def _prep_body(d_ref, r_ref, ec_ref):
    d = d_ref[...]
    r = r_ref[...]
    dn = jnp.maximum(jnp.sqrt(jnp.sum(d * d, axis=1, keepdims=True)), _EPS)
    rn = jnp.maximum(jnp.sqrt(jnp.sum(r * r, axis=1, keepdims=True)), _EPS)
    e = r / rn - d / dn
    c = jnp.sum(e * e, axis=1, keepdims=True)
    pad = jnp.zeros((CPAD, EW - D - 1), jnp.float32)
    ec_ref[...] = jnp.concatenate([e, c, pad], axis=1)


@functools.partial(
    pl.kernel,
    out_type=jax.ShapeDtypeStruct((B,), jnp.float32),
    mesh=plsc.VectorSubcoreMesh(core_axis_name="c", subcore_axis_name="s"),
    scratch_types=[
        pltpu.VMEM((RPW,), jnp.int32),
        pltpu.VMEM((RPW,), jnp.int32),
        pltpu.VMEM((RPW,), jnp.int32),
        pltpu.VMEM((CH, 8, D), jnp.float32),
        pltpu.VMEM((CH, EW), jnp.float32),
        pltpu.VMEM((RPW,), jnp.float32),
        pltpu.SemaphoreType.DMA,
        pltpu.SemaphoreType.DMA,
    ],
    compiler_params=pltpu.CompilerParams(needs_layout_passes=False,
                                         use_tc_tiling_on_sc=True),
)
def _sc_main(gene_idx, drug_idx, gt3, ec_tab, out_hbm,
             gidx_v, gtidx_v, didx_v, g_v, e_v, out_v, sem, sem2):
    wid = lax.axis_index("s") * NC + lax.axis_index("c")
    base = wid * RPW
    pltpu.sync_copy(gene_idx.at[pl.ds(base, RPW)], gidx_v)
    pltpu.sync_copy(drug_idx.at[pl.ds(base, RPW)], didx_v)

    def shft(t, carry):
        gtidx_v[pl.ds(t * L, L)] = gidx_v[pl.ds(t * L, L)] >> 3
        return carry

    lax.fori_loop(0, RPW // L, shft, 0)

    iota = lax.iota(jnp.int32, L)

    for p in range(NP):
        pbase = p * CH
        cp_e = pltpu.async_copy(ec_tab.at[didx_v.at[pl.ds(pbase, CH)]],
                                e_v, sem2)

        def fire(t, carry):
            tbv = gtidx_v[pl.ds(pbase + t * L, L)]
            for k in range(L):
                pltpu.async_copy(gt3.at[tbv[k]], g_v.at[t * L + k], sem)
            return carry

        lax.fori_loop(0, CH // L, fire, 0)
        for k in range(CH // L):
            pltpu.make_async_copy(gt3.at[pl.ds(0, L)],
                                  g_v.at[pl.ds(k * L, L)], sem).wait()
        cp_e.wait()

        def grp(t, carry):
            rows = t * L + iota
            subs = gidx_v[pl.ds(pbase + t * L, L)] & jnp.int32(7)
            gg = jnp.zeros((L,), jnp.float32)
            ge = jnp.zeros((L,), jnp.float32)
            for j in range(D):
                dcol = (iota + j) & (D - 1)
                g = plsc.load_gather(g_v, [rows, subs, dcol])
                e = plsc.load_gather(e_v, [rows, dcol])
                gg = gg + g * g
                ge = ge + g * e
            cc = plsc.load_gather(e_v, [rows, jnp.full((L,), D, jnp.int32)])
            s = jnp.maximum(gg * _rsqrt_vec(gg), jnp.float32(_EPS))
            inv = jnp.float32(1.0) / s
            tot = jnp.maximum(
                gg * inv * inv + (jnp.float32(2.0) * inv) * ge + cc,
                jnp.float32(0.0))
            out_v[pl.ds(pbase + t * L, L)] = (jnp.float32(0.0)
                                              - tot * _rsqrt_vec(tot))
            return carry

        lax.fori_loop(0, CH // L, grp, 0)

    pltpu.sync_copy(out_v, out_hbm.at[pl.ds(base, RPW)])


def kernel(gene_idx, drug_idx, gene_table, drug_table, relation):
    gene_idx = gene_idx.astype(jnp.int32)
    drug_idx = drug_idx.astype(jnp.int32)
    nd = drug_table.shape[0]
    d_pad = jnp.pad(drug_table, ((0, CPAD - nd), (0, 0)), constant_values=1.0)
    ec_tab = pl.pallas_call(
        _prep_body,
        out_shape=jax.ShapeDtypeStruct((CPAD, EW), jnp.float32),
    )(d_pad, relation.reshape(1, D))
    gt3 = gene_table.reshape(gene_table.shape[0] // 8, 8, D)
    return _sc_main(gene_idx, drug_idx, gt3, ec_tab)


# speedup vs baseline: 2.2645x; 1.0152x over previous
import functools

import jax
import jax.numpy as jnp
from jax import lax
from jax.experimental import pallas as pl
from jax.experimental.pallas import tpu as pltpu
from jax.experimental.pallas import tpu_sc as plsc

NC, NS, L = 2, 16, 16
NW = NC * NS
B = 16384
D = 64
RPW = B // NW
CH = 32
NP = RPW // CH
CPAD = 1024
EW = 128

_EPS = 1e-12


def _rsqrt_vec(x):
    i = plsc.bitcast(x, jnp.int32)
    y = plsc.bitcast(jnp.int32(0x5F3759DF) - (i >> 1), jnp.float32)
    hx = x * jnp.float32(0.5)
    for _ in range(3):
        y = y * (jnp.float32(1.5) - (hx * y) * y)
    return y


def _prep_body(d_ref, r_ref, ec_ref):
    d = d_ref[...]
    r = r_ref[...]
    dn = jnp.maximum(jnp.sqrt(jnp.sum(d * d, axis=1, keepdims=True)), _EPS)
    rn = jnp.maximum(jnp.sqrt(jnp.sum(r * r, axis=1, keepdims=True)), _EPS)
    e = r / rn - d / dn
    c = jnp.sum(e * e, axis=1, keepdims=True)
    pad = jnp.zeros((CPAD, EW - D - 1), jnp.float32)
    ec_ref[...] = jnp.concatenate([e, c, pad], axis=1)


@functools.partial(
    pl.kernel,
    out_type=jax.ShapeDtypeStruct((B,), jnp.float32),
    mesh=plsc.VectorSubcoreMesh(core_axis_name="c", subcore_axis_name="s"),
    scratch_types=[
        pltpu.VMEM((RPW,), jnp.int32),
        pltpu.VMEM((RPW,), jnp.int32),
        pltpu.VMEM((RPW,), jnp.int32),
        pltpu.VMEM((CH, 8, D), jnp.float32),
        pltpu.VMEM((CH, 8, D), jnp.float32),
        pltpu.VMEM((CH, EW), jnp.float32),
        pltpu.VMEM((CH, EW), jnp.float32),
        pltpu.VMEM((RPW,), jnp.float32),
        pltpu.SemaphoreType.DMA,
        pltpu.SemaphoreType.DMA,
        pltpu.SemaphoreType.DMA,
        pltpu.SemaphoreType.DMA,
    ],
    compiler_params=pltpu.CompilerParams(needs_layout_passes=False,
                                         use_tc_tiling_on_sc=True),
)
def _sc_main(gene_idx, drug_idx, gt3, ec_tab, out_hbm,
             gidx_v, gtidx_v, didx_v, g_v0, g_v1, e_v0, e_v1, out_v,
             sg0, sg1, se0, se1):
    wid = lax.axis_index("s") * NC + lax.axis_index("c")
    base = wid * RPW
    pltpu.sync_copy(gene_idx.at[pl.ds(base, RPW)], gidx_v)
    pltpu.sync_copy(drug_idx.at[pl.ds(base, RPW)], didx_v)

    def shft(t, carry):
        gtidx_v[pl.ds(t * L, L)] = gidx_v[pl.ds(t * L, L)] >> 3
        return carry

    lax.fori_loop(0, RPW // L, shft, 0)

    iota = lax.iota(jnp.int32, L)

    gbufs = (g_v0, g_v1)
    ebufs = (e_v0, e_v1)
    gsems = (sg0, sg1)
    esems = (se0, se1)

    def fire(p):
        pb = p * CH
        gbuf, ebuf = gbufs[p % 2], ebufs[p % 2]
        ecp = pltpu.async_copy(ec_tab.at[didx_v.at[pl.ds(pb, CH)]],
                               ebuf, esems[p % 2])
        for k in range(CH // L):
            tbv = gtidx_v[pl.ds(pb + k * L, L)]
            for i in range(L):
                pltpu.async_copy(gt3.at[tbv[i]], gbuf.at[k * L + i],
                                 gsems[p % 2])
        return ecp

    def drain(p, ecp):
        gbuf = gbufs[p % 2]
        for k in range(CH // L):
            pltpu.make_async_copy(gt3.at[pl.ds(0, L)],
                                  gbuf.at[pl.ds(k * L, L)],
                                  gsems[p % 2]).wait()
        ecp.wait()

    def compute(p):
        pb = p * CH
        gbuf, ebuf = gbufs[p % 2], ebufs[p % 2]

        def grp(t, carry):
            rows = t * L + iota
            subs = gidx_v[pl.ds(pb + t * L, L)] & jnp.int32(7)
            gg = jnp.zeros((L,), jnp.float32)
            ge = jnp.zeros((L,), jnp.float32)
            for j in range(D):
                dcol = (iota + j) & (D - 1)
                g = plsc.load_gather(gbuf, [rows, subs, dcol])
                e = plsc.load_gather(ebuf, [rows, dcol])
                gg = gg + g * g
                ge = ge + g * e
            cc = plsc.load_gather(ebuf, [rows, jnp.full((L,), D, jnp.int32)])
            s = jnp.maximum(gg * _rsqrt_vec(gg), jnp.float32(_EPS))
            inv = jnp.float32(1.0) / s
            tot = jnp.maximum(
                gg * inv * inv + (jnp.float32(2.0) * inv) * ge + cc,
                jnp.float32(0.0))
            out_v[pl.ds(pb + t * L, L)] = (jnp.float32(0.0)
                                           - tot * _rsqrt_vec(tot))
            return carry

        lax.fori_loop(0, CH // L, grp, 0)

    ecps = {0: fire(0)}
    for p in range(NP):
        if p + 1 < NP:
            ecps[p + 1] = fire(p + 1)
        drain(p, ecps.pop(p))
        compute(p)

    pltpu.sync_copy(out_v, out_hbm.at[pl.ds(base, RPW)])


def kernel(gene_idx, drug_idx, gene_table, drug_table, relation):
    gene_idx = gene_idx.astype(jnp.int32)
    drug_idx = drug_idx.astype(jnp.int32)
    nd = drug_table.shape[0]
    d_pad = jnp.pad(drug_table, ((0, CPAD - nd), (0, 0)), constant_values=1.0)
    ec_tab = pl.pallas_call(
        _prep_body,
        out_shape=jax.ShapeDtypeStruct((CPAD, EW), jnp.float32),
    )(d_pad, relation.reshape(1, D))
    gt3 = gene_table.reshape(gene_table.shape[0] // 8, 8, D)
    return _sc_main(gene_idx, drug_idx, gt3, ec_tab)
